# baseline (speedup 1.0000x reference)
import functools
import jax
import jax.numpy as jnp
from jax import lax
from jax.experimental import pallas as pl
from jax.experimental.pallas import tpu as pltpu
from jax.experimental.pallas import tpu_sc as plsc

NC = 2
NS = 16
CHUNK = 128


def _make_sc_call(E, V, D):
    HALF = V // 2
    ROWS_PT = -(-HALF // (8 * NS)) * 8
    ACC_ROWS = NS * ROWS_PT
    GARBAGE = HALF
    assert ACC_ROWS >= HALF + 16 and E % CHUNK == 0
    N_VREG = CHUNK // 16

    mesh = plsc.VectorSubcoreMesh(core_axis_name="c", subcore_axis_name="s")

    @functools.partial(
        pl.kernel,
        mesh=mesh,
        compiler_params=pltpu.CompilerParams(use_tc_tiling_on_sc=False),
        out_type=jax.ShapeDtypeStruct((NC * ACC_ROWS, D), jnp.float32),
        scratch_types=[
            pltpu.VMEM((CHUNK, D), jnp.float32),
            pltpu.VMEM((CHUNK,), jnp.int32),
            pltpu.VMEM((CHUNK,), jnp.int32),
            pltpu.VMEM((16,), jnp.int32),
            pltpu.VMEM_SHARED((ACC_ROWS, D), jnp.float32),
        ],
    )
    def sc_call(src_h, idx_h, tbl_h, zer_h, out_h, sbuf, ibuf, i2buf, tblv,
                acc):
        c = lax.axis_index("c")
        s = lax.axis_index("s")
        wid = c * NS + s
        pltpu.sync_copy(tbl_h.at[pl.ds(wid * 16, 16)], tblv)
        tvec = tblv[pl.ds(0, 16)]
        start = tvec[0]
        end = tvec[1]
        start_al = tvec[2]
        nch = tvec[3]
        node_base = c * HALF

        pltpu.sync_copy(zer_h, acc.at[pl.ds(s * ROWS_PT, ROWS_PT)])
        plsc.subcore_barrier()

        e_cap = E - CHUNK
        iota = lax.iota(jnp.int32, 16)
        garbage = GARBAGE + (iota & 15)

        def body(i, carry):
            e0u = start_al + i * CHUNK
            e0 = pl.multiple_of(jnp.minimum(e0u, e_cap), 8)
            pltpu.sync_copy(src_h.at[pl.ds(e0, CHUNK)], sbuf)
            pltpu.sync_copy(idx_h.at[pl.ds(e0, CHUNK)], ibuf)
            lo_ok = jnp.maximum(start, e0u)
            for v in range(N_VREG):
                pos = iota + (e0 + v * 16)
                iv = ibuf[pl.ds(v * 16, 16)]
                ok = (pos >= lo_ok) & (pos < end)
                adj = jnp.where(ok, iv - node_base, garbage)
                i2buf[pl.ds(v * 16, 16)] = adj
            pltpu.sync_copy(sbuf, acc.at[i2buf], add=True)
            return carry

        lax.fori_loop(0, nch, body, 0)
        plsc.subcore_barrier()
        pltpu.sync_copy(acc.at[pl.ds(s * ROWS_PT, ROWS_PT)],
                        out_h.at[pl.ds(wid * ROWS_PT, ROWS_PT)])

    return sc_call, ACC_ROWS, ROWS_PT


def kernel(src, index, pos):
    E, R0, R1 = src.shape
    D = R0 * R1
    V = pos.shape[0]
    idx = index.astype(jnp.int32)
    HALF = V // 2
    sc_call, ACC_ROWS, ROWS_PT = _make_sc_call(E, V, D)

    p = jnp.searchsorted(idx, jnp.int32(HALF), side="left").astype(jnp.int32)
    w = jnp.arange(NC * NS, dtype=jnp.int32)
    c = w // NS
    s = w % NS
    lo = jnp.where(c == 0, jnp.int32(0), p)
    hi = jnp.where(c == 0, p, jnp.int32(E))
    per = (hi - lo + NS - 1) // NS
    start = jnp.minimum(lo + s * per, hi)
    end = jnp.minimum(start + per, hi)
    start_al = (start // 8) * 8
    nch = (end - start_al + CHUNK - 1) // CHUNK
    tbl = jnp.stack([start, end, start_al, nch], axis=1).astype(jnp.int32)
    tbl = jnp.pad(tbl, ((0, 0), (0, 12))).reshape(-1)
    zer = jnp.zeros((ROWS_PT, D), jnp.float32)

    out = sc_call(src.reshape(E, D), idx, tbl, zer)
    out = jnp.concatenate([out[:HALF], out[ACC_ROWS:ACC_ROWS + HALF]], axis=0)
    return out.reshape(V, R0, R1)


# speedup vs baseline: 1.2905x; 1.2905x over previous
import functools
import jax
import jax.numpy as jnp
from jax import lax
from jax.experimental import pallas as pl
from jax.experimental.pallas import tpu as pltpu
from jax.experimental.pallas import tpu_sc as plsc

NC = 2
NS = 16
CHUNK = 128
NBUF = 2


def _make_sc_call(E, V, D):
    HALF = V // 2
    ROWS_PT = -(-HALF // (8 * NS)) * 8
    ACC_ROWS = NS * ROWS_PT
    GARBAGE = HALF
    assert ACC_ROWS >= HALF + 16 and E % CHUNK == 0
    N_VREG = CHUNK // 16

    mesh = plsc.VectorSubcoreMesh(core_axis_name="c", subcore_axis_name="s")

    @functools.partial(
        pl.kernel,
        mesh=mesh,
        compiler_params=pltpu.CompilerParams(use_tc_tiling_on_sc=False),
        out_type=jax.ShapeDtypeStruct((NC * ACC_ROWS, D), jnp.float32),
        scratch_types=[
            pltpu.VMEM((NBUF * CHUNK, D), jnp.float32),
            pltpu.VMEM((NBUF * CHUNK,), jnp.int32),
            pltpu.VMEM((NBUF * CHUNK,), jnp.int32),
            pltpu.VMEM((16,), jnp.int32),
            pltpu.VMEM_SHARED((ACC_ROWS, D), jnp.float32),
            pltpu.SemaphoreType.DMA((NBUF,)),
            pltpu.SemaphoreType.DMA((NBUF,)),
        ],
    )
    def sc_call(src_h, idx_h, tbl_h, zer_h, out_h, sbuf, ibuf, i2buf, tblv,
                acc, ssem, isem):
        c = lax.axis_index("c")
        s = lax.axis_index("s")
        wid = c * NS + s
        pltpu.sync_copy(tbl_h.at[pl.ds(wid * 16, 16)], tblv)
        tvec = tblv[pl.ds(0, 16)]
        start = tvec[0]
        end = tvec[1]
        start_al = tvec[2]
        n_outer = tvec[3]
        node_base = c * HALF

        e_cap = E - CHUNK
        iota = lax.iota(jnp.int32, 16)
        garbage = GARBAGE + (iota & 15)

        def chunk_off(i):
            e0u = start_al + i * CHUNK
            return pl.multiple_of(jnp.minimum(e0u, e_cap), 8), e0u

        for b in range(NBUF):
            e0, _ = chunk_off(jnp.int32(b))
            pltpu.async_copy(src_h.at[pl.ds(e0, CHUNK)],
                             sbuf.at[pl.ds(b * CHUNK, CHUNK)], ssem.at[b])
            pltpu.async_copy(idx_h.at[pl.ds(e0, CHUNK)],
                             ibuf.at[pl.ds(b * CHUNK, CHUNK)], isem.at[b])

        pltpu.sync_copy(zer_h, acc.at[pl.ds(s * ROWS_PT, ROWS_PT)])
        plsc.subcore_barrier()

        def outer(o, carry):
            for b in range(NBUF):
                i = o * NBUF + b
                e0, e0u = chunk_off(i)
                pltpu.make_async_copy(src_h.at[pl.ds(0, CHUNK)],
                                      sbuf.at[pl.ds(b * CHUNK, CHUNK)],
                                      ssem.at[b]).wait()
                pltpu.make_async_copy(idx_h.at[pl.ds(0, CHUNK)],
                                      ibuf.at[pl.ds(b * CHUNK, CHUNK)],
                                      isem.at[b]).wait()
                lo_ok = jnp.maximum(start, e0u)
                for v in range(N_VREG):
                    pos = iota + (e0 + v * 16)
                    iv = ibuf[pl.ds(b * CHUNK + v * 16, 16)]
                    ok = (pos >= lo_ok) & (pos < end)
                    adj = jnp.where(ok, iv - node_base, garbage)
                    i2buf[pl.ds(b * CHUNK + v * 16, 16)] = adj
                pltpu.sync_copy(sbuf.at[pl.ds(b * CHUNK, CHUNK)],
                                acc.at[i2buf.at[pl.ds(b * CHUNK, CHUNK)]],
                                add=True)
                e0n, _ = chunk_off(i + NBUF)
                pltpu.async_copy(src_h.at[pl.ds(e0n, CHUNK)],
                                 sbuf.at[pl.ds(b * CHUNK, CHUNK)], ssem.at[b])
                pltpu.async_copy(idx_h.at[pl.ds(e0n, CHUNK)],
                                 ibuf.at[pl.ds(b * CHUNK, CHUNK)], isem.at[b])
            return carry

        lax.fori_loop(0, n_outer, outer, 0)

        for b in range(NBUF):
            pltpu.make_async_copy(src_h.at[pl.ds(0, CHUNK)],
                                  sbuf.at[pl.ds(b * CHUNK, CHUNK)],
                                  ssem.at[b]).wait()
            pltpu.make_async_copy(idx_h.at[pl.ds(0, CHUNK)],
                                  ibuf.at[pl.ds(b * CHUNK, CHUNK)],
                                  isem.at[b]).wait()

        plsc.subcore_barrier()
        pltpu.sync_copy(acc.at[pl.ds(s * ROWS_PT, ROWS_PT)],
                        out_h.at[pl.ds(wid * ROWS_PT, ROWS_PT)])

    return sc_call, ACC_ROWS, ROWS_PT


def kernel(src, index, pos):
    E, R0, R1 = src.shape
    D = R0 * R1
    V = pos.shape[0]
    idx = index.astype(jnp.int32)
    HALF = V // 2
    sc_call, ACC_ROWS, ROWS_PT = _make_sc_call(E, V, D)

    p = jnp.searchsorted(idx, jnp.int32(HALF), side="left").astype(jnp.int32)
    w = jnp.arange(NC * NS, dtype=jnp.int32)
    c = w // NS
    s = w % NS
    lo = jnp.where(c == 0, jnp.int32(0), p)
    hi = jnp.where(c == 0, p, jnp.int32(E))
    per = (hi - lo + NS - 1) // NS
    start = jnp.minimum(lo + s * per, hi)
    end = jnp.minimum(start + per, hi)
    start_al = (start // 8) * 8
    nch = (end - start_al + CHUNK - 1) // CHUNK
    n_outer = (nch + NBUF - 1) // NBUF
    tbl = jnp.stack([start, end, start_al, n_outer], axis=1).astype(jnp.int32)
    tbl = jnp.pad(tbl, ((0, 0), (0, 12))).reshape(-1)
    zer = jnp.zeros((ROWS_PT, D), jnp.float32)

    out = sc_call(src.reshape(E, D), idx, tbl, zer)
    out = jnp.concatenate([out[:HALF], out[ACC_ROWS:ACC_ROWS + HALF]], axis=0)
    return out.reshape(V, R0, R1)


# speedup vs baseline: 1.3501x; 1.0462x over previous
import functools
import jax
import jax.numpy as jnp
from jax import lax
from jax.experimental import pallas as pl
from jax.experimental.pallas import tpu as pltpu
from jax.experimental.pallas import tpu_sc as plsc

NC = 2
NS = 16
CHUNK = 128
NBUF = 2


def _make_sc_call(E, V, D):
    HALF = V // 2
    ROWS_PT = -(-HALF // (8 * NS)) * 8
    ACC_ROWS = NS * ROWS_PT
    GARBAGE = HALF
    G8 = HALF // 8
    BASE_G = G8 // NS
    N_EXTRA = G8 - BASE_G * NS
    BASE_ROWS = BASE_G * 8
    assert ACC_ROWS >= HALF + 16 and E % CHUNK == 0 and HALF % 8 == 0
    N_VREG = CHUNK // 16

    mesh = plsc.VectorSubcoreMesh(core_axis_name="c", subcore_axis_name="s")

    @functools.partial(
        pl.kernel,
        mesh=mesh,
        compiler_params=pltpu.CompilerParams(use_tc_tiling_on_sc=False),
        out_type=jax.ShapeDtypeStruct((V, D), jnp.float32),
        scratch_types=[
            pltpu.VMEM((NBUF * CHUNK, D), jnp.float32),
            pltpu.VMEM((NBUF * CHUNK,), jnp.int32),
            pltpu.VMEM((NBUF * CHUNK,), jnp.int32),
            pltpu.VMEM((16,), jnp.int32),
            pltpu.VMEM_SHARED((ACC_ROWS, D), jnp.float32),
            pltpu.SemaphoreType.DMA((NBUF,)),
            pltpu.SemaphoreType.DMA((NBUF,)),
        ],
    )
    def sc_call(src_h, idx_h, tbl_h, zer_h, out_h, sbuf, ibuf, i2buf, tblv,
                acc, ssem, isem):
        c = lax.axis_index("c")
        s = lax.axis_index("s")
        wid = c * NS + s
        pltpu.sync_copy(tbl_h.at[pl.ds(wid * 16, 16)], tblv)
        tvec = tblv[pl.ds(0, 16)]
        start = tvec[0]
        end = tvec[1]
        start_al = tvec[2]
        n_outer = tvec[3]
        rowbase = tvec[4]
        node_base = c * HALF

        e_cap = E - CHUNK
        iota = lax.iota(jnp.int32, 16)
        garbage = GARBAGE + (iota & 15)

        def chunk_off(i):
            e0u = start_al + i * CHUNK
            return pl.multiple_of(jnp.minimum(e0u, e_cap), 8), e0u

        for b in range(NBUF):
            e0, _ = chunk_off(jnp.int32(b))
            pltpu.async_copy(src_h.at[pl.ds(e0, CHUNK)],
                             sbuf.at[pl.ds(b * CHUNK, CHUNK)], ssem.at[b])
            pltpu.async_copy(idx_h.at[pl.ds(e0, CHUNK)],
                             ibuf.at[pl.ds(b * CHUNK, CHUNK)], isem.at[b])

        pltpu.sync_copy(zer_h, acc.at[pl.ds(s * ROWS_PT, ROWS_PT)])
        plsc.subcore_barrier()

        def outer(o, carry):
            for b in range(NBUF):
                i = o * NBUF + b
                e0, e0u = chunk_off(i)
                pltpu.make_async_copy(src_h.at[pl.ds(0, CHUNK)],
                                      sbuf.at[pl.ds(b * CHUNK, CHUNK)],
                                      ssem.at[b]).wait()
                pltpu.make_async_copy(idx_h.at[pl.ds(0, CHUNK)],
                                      ibuf.at[pl.ds(b * CHUNK, CHUNK)],
                                      isem.at[b]).wait()
                lo_ok = jnp.maximum(start, e0u)
                for v in range(N_VREG):
                    pos = iota + (e0 + v * 16)
                    iv = ibuf[pl.ds(b * CHUNK + v * 16, 16)]
                    ok = (pos >= lo_ok) & (pos < end)
                    adj = jnp.where(ok, iv - node_base, garbage)
                    i2buf[pl.ds(b * CHUNK + v * 16, 16)] = adj
                pltpu.sync_copy(sbuf.at[pl.ds(b * CHUNK, CHUNK)],
                                acc.at[i2buf.at[pl.ds(b * CHUNK, CHUNK)]],
                                add=True)
                e0n, _ = chunk_off(i + NBUF)
                pltpu.async_copy(src_h.at[pl.ds(e0n, CHUNK)],
                                 sbuf.at[pl.ds(b * CHUNK, CHUNK)], ssem.at[b])
                pltpu.async_copy(idx_h.at[pl.ds(e0n, CHUNK)],
                                 ibuf.at[pl.ds(b * CHUNK, CHUNK)], isem.at[b])
            return carry

        lax.fori_loop(0, n_outer, outer, 0)

        for b in range(NBUF):
            pltpu.make_async_copy(src_h.at[pl.ds(0, CHUNK)],
                                  sbuf.at[pl.ds(b * CHUNK, CHUNK)],
                                  ssem.at[b]).wait()
            pltpu.make_async_copy(idx_h.at[pl.ds(0, CHUNK)],
                                  ibuf.at[pl.ds(b * CHUNK, CHUNK)],
                                  isem.at[b]).wait()

        plsc.subcore_barrier()
        pltpu.sync_copy(acc.at[pl.ds(rowbase, BASE_ROWS)],
                        out_h.at[pl.ds(node_base + rowbase, BASE_ROWS)])

        @pl.when(s < N_EXTRA)
        def _():
            pltpu.sync_copy(
                acc.at[pl.ds(rowbase + BASE_ROWS, 8)],
                out_h.at[pl.ds(node_base + rowbase + BASE_ROWS, 8)])

    return sc_call, ROWS_PT, BASE_ROWS, N_EXTRA


def kernel(src, index, pos):
    E, R0, R1 = src.shape
    D = R0 * R1
    V = pos.shape[0]
    idx = index.astype(jnp.int32)
    HALF = V // 2
    sc_call, ROWS_PT, BASE_ROWS, N_EXTRA = _make_sc_call(E, V, D)

    p = jnp.searchsorted(idx, jnp.int32(HALF), side="left").astype(jnp.int32)
    w = jnp.arange(NC * NS, dtype=jnp.int32)
    c = w // NS
    s = w % NS
    lo = jnp.where(c == 0, jnp.int32(0), p)
    hi = jnp.where(c == 0, p, jnp.int32(E))
    per = (hi - lo + NS - 1) // NS
    start = jnp.minimum(lo + s * per, hi)
    end = jnp.minimum(start + per, hi)
    start_al = (start // 8) * 8
    nch = (end - start_al + CHUNK - 1) // CHUNK
    n_outer = (nch + NBUF - 1) // NBUF
    rowbase = jnp.where(s < N_EXTRA, s * (BASE_ROWS + 8),
                        N_EXTRA * (BASE_ROWS + 8) + (s - N_EXTRA) * BASE_ROWS)
    tbl = jnp.stack([start, end, start_al, n_outer, rowbase],
                    axis=1).astype(jnp.int32)
    tbl = jnp.pad(tbl, ((0, 0), (0, 11))).reshape(-1)
    zer = jnp.zeros((ROWS_PT, D), jnp.float32)

    out = sc_call(src.reshape(E, D), idx, tbl, zer)
    return out.reshape(V, R0, R1)


# speedup vs baseline: 1.4233x; 1.0542x over previous
import functools
import jax
import jax.numpy as jnp
from jax import lax
from jax.experimental import pallas as pl
from jax.experimental.pallas import tpu as pltpu
from jax.experimental.pallas import tpu_sc as plsc

NC = 2
NS = 16
CHUNK = 128
NBUF = 3


def _make_sc_call(E, V, D):
    HALF = V // 2
    ROWS_PT = -(-HALF // (8 * NS)) * 8
    ACC_ROWS = NS * ROWS_PT
    GARBAGE = HALF
    G8 = HALF // 8
    BASE_G = G8 // NS
    N_EXTRA = G8 - BASE_G * NS
    BASE_ROWS = BASE_G * 8
    assert ACC_ROWS >= HALF + 16 and E % CHUNK == 0 and HALF % 8 == 0
    N_VREG = CHUNK // 16

    mesh = plsc.VectorSubcoreMesh(core_axis_name="c", subcore_axis_name="s")

    @functools.partial(
        pl.kernel,
        mesh=mesh,
        compiler_params=pltpu.CompilerParams(use_tc_tiling_on_sc=False),
        out_type=jax.ShapeDtypeStruct((V, D), jnp.float32),
        scratch_types=[
            pltpu.VMEM((NBUF * CHUNK, D), jnp.float32),
            pltpu.VMEM((NBUF * CHUNK,), jnp.int32),
            pltpu.VMEM((NBUF * CHUNK,), jnp.int32),
            pltpu.VMEM((16,), jnp.int32),
            pltpu.VMEM_SHARED((ACC_ROWS, D), jnp.float32),
            pltpu.SemaphoreType.DMA((NBUF,)),
            pltpu.SemaphoreType.DMA((NBUF,)),
        ],
    )
    def sc_call(src_h, idx_h, tbl_h, zer_h, out_h, sbuf, ibuf, i2buf, tblv,
                acc, ssem, isem):
        c = lax.axis_index("c")
        s = lax.axis_index("s")
        wid = c * NS + s
        pltpu.sync_copy(tbl_h.at[pl.ds(wid * 16, 16)], tblv)
        tvec = tblv[pl.ds(0, 16)]
        start = tvec[0]
        end = tvec[1]
        start_al = tvec[2]
        n_outer = tvec[3]
        rowbase = tvec[4]
        node_base = c * HALF

        e_cap = E - CHUNK
        iota = lax.iota(jnp.int32, 16)
        garbage = GARBAGE + (iota & 15)

        def chunk_off(i):
            e0u = start_al + i * CHUNK
            return pl.multiple_of(jnp.minimum(e0u, e_cap), 8), e0u

        for b in range(NBUF):
            e0, _ = chunk_off(jnp.int32(b))
            pltpu.async_copy(src_h.at[pl.ds(e0, CHUNK)],
                             sbuf.at[pl.ds(b * CHUNK, CHUNK)], ssem.at[b])
            pltpu.async_copy(idx_h.at[pl.ds(e0, CHUNK)],
                             ibuf.at[pl.ds(b * CHUNK, CHUNK)], isem.at[b])

        pltpu.sync_copy(zer_h, acc.at[pl.ds(s * ROWS_PT, ROWS_PT)])
        plsc.subcore_barrier()

        def outer(o, carry):
            for b in range(NBUF):
                i = o * NBUF + b
                e0, e0u = chunk_off(i)
                pltpu.make_async_copy(src_h.at[pl.ds(0, CHUNK)],
                                      sbuf.at[pl.ds(b * CHUNK, CHUNK)],
                                      ssem.at[b]).wait()
                pltpu.make_async_copy(idx_h.at[pl.ds(0, CHUNK)],
                                      ibuf.at[pl.ds(b * CHUNK, CHUNK)],
                                      isem.at[b]).wait()
                lo_ok = jnp.maximum(start, e0u)
                for v in range(N_VREG):
                    pos = iota + (e0 + v * 16)
                    iv = ibuf[pl.ds(b * CHUNK + v * 16, 16)]
                    ok = (pos >= lo_ok) & (pos < end)
                    adj = jnp.where(ok, iv - node_base, garbage)
                    i2buf[pl.ds(b * CHUNK + v * 16, 16)] = adj
                pltpu.sync_copy(sbuf.at[pl.ds(b * CHUNK, CHUNK)],
                                acc.at[i2buf.at[pl.ds(b * CHUNK, CHUNK)]],
                                add=True)
                e0n, _ = chunk_off(i + NBUF)
                pltpu.async_copy(src_h.at[pl.ds(e0n, CHUNK)],
                                 sbuf.at[pl.ds(b * CHUNK, CHUNK)], ssem.at[b])
                pltpu.async_copy(idx_h.at[pl.ds(e0n, CHUNK)],
                                 ibuf.at[pl.ds(b * CHUNK, CHUNK)], isem.at[b])
            return carry

        lax.fori_loop(0, n_outer, outer, 0)

        for b in range(NBUF):
            pltpu.make_async_copy(src_h.at[pl.ds(0, CHUNK)],
                                  sbuf.at[pl.ds(b * CHUNK, CHUNK)],
                                  ssem.at[b]).wait()
            pltpu.make_async_copy(idx_h.at[pl.ds(0, CHUNK)],
                                  ibuf.at[pl.ds(b * CHUNK, CHUNK)],
                                  isem.at[b]).wait()

        plsc.subcore_barrier()
        pltpu.sync_copy(acc.at[pl.ds(rowbase, BASE_ROWS)],
                        out_h.at[pl.ds(node_base + rowbase, BASE_ROWS)])

        @pl.when(s < N_EXTRA)
        def _():
            pltpu.sync_copy(
                acc.at[pl.ds(rowbase + BASE_ROWS, 8)],
                out_h.at[pl.ds(node_base + rowbase + BASE_ROWS, 8)])

    return sc_call, ROWS_PT, BASE_ROWS, N_EXTRA


def kernel(src, index, pos):
    E, R0, R1 = src.shape
    D = R0 * R1
    V = pos.shape[0]
    idx = index.astype(jnp.int32)
    HALF = V // 2
    sc_call, ROWS_PT, BASE_ROWS, N_EXTRA = _make_sc_call(E, V, D)

    p = jnp.searchsorted(idx, jnp.int32(HALF), side="left").astype(jnp.int32)
    w = jnp.arange(NC * NS, dtype=jnp.int32)
    c = w // NS
    s = w % NS
    lo = jnp.where(c == 0, jnp.int32(0), p)
    hi = jnp.where(c == 0, p, jnp.int32(E))
    per = (hi - lo + NS - 1) // NS
    start = jnp.minimum(lo + s * per, hi)
    end = jnp.minimum(start + per, hi)
    start_al = (start // 8) * 8
    nch = (end - start_al + CHUNK - 1) // CHUNK
    n_outer = (nch + NBUF - 1) // NBUF
    rowbase = jnp.where(s < N_EXTRA, s * (BASE_ROWS + 8),
                        N_EXTRA * (BASE_ROWS + 8) + (s - N_EXTRA) * BASE_ROWS)
    tbl = jnp.stack([start, end, start_al, n_outer, rowbase],
                    axis=1).astype(jnp.int32)
    tbl = jnp.pad(tbl, ((0, 0), (0, 11))).reshape(-1)
    zer = jnp.zeros((ROWS_PT, D), jnp.float32)

    out = sc_call(src.reshape(E, D), idx, tbl, zer)
    return out.reshape(V, R0, R1)
